# baseline (speedup 1.0000x reference)
import math

import jax
import jax.numpy as jnp
from jax import lax
from jax.experimental import pallas as pl
from jax.experimental.pallas import tpu as pltpu
from jax.experimental.pallas import tpu_sc as plsc

VOCAB_SIZE = 1000000
EMB_D = 240
ROW_PHYS = 256
SEQ = 200
BATCH = 4096

NUM_CORES = 2
NUM_SUBCORES = 16
NUM_WORKERS = NUM_CORES * NUM_SUBCORES
BTOK = BATCH // NUM_WORKERS
W_A = 128
W_B = EMB_D - W_A
GROUPS = BTOK // 16

_SCALE = math.sqrt(EMB_D)


def _emb_body(tok_hbm, table_hbm, out_hbm,
              idx_stage, idx_ring, bg0, bg1, tr_a, tr_b,
              sem_g0, sem_g1, sem_sa, sem_sb):
    buf_g = (bg0, bg1)
    sem_g = (sem_g0, sem_g1)

    wid = lax.axis_index("s") * NUM_CORES + lax.axis_index("c")
    b0 = pl.multiple_of(wid * BTOK, 128)

    tvecs = [lax.iota(jnp.int32, 16) + (g * 16) for g in range(GROUPS)]

    pltpu.sync_copy(tok_hbm.at[pl.ds(b0, BTOK)], idx_stage)

    def build_idx(s, p):
        svec = jnp.full((16,), s, jnp.int32)
        for g in range(GROUPS):
            idx_ring[p, pl.ds(g * 16, 16)] = \
                plsc.load_gather(idx_stage, [tvecs[g], svec])

    def fire_gather(p):
        pltpu.async_copy(
            table_hbm.at[idx_ring.at[p], pl.ds(0, ROW_PHYS)],
            buf_g[p], sem_g[p])

    def wait_gather(p):
        pltpu.make_async_copy(
            table_hbm.at[idx_ring.at[p], pl.ds(0, ROW_PHYS)],
            buf_g[p], sem_g[p]).wait()

    def store_dst(s, half):
        if half == 0:
            return out_hbm.at[pl.ds(s, 1), pl.ds(0, W_A), pl.ds(b0, BTOK)]
        return out_hbm.at[pl.ds(s, 1), pl.ds(W_A, W_B), pl.ds(b0, BTOK)]

    def transpose_scale(p, half):
        dst = tr_a if half == 0 else tr_b
        width = W_A if half == 0 else W_B
        cbase = 0 if half == 0 else W_A

        @pl.loop(0, width, unroll=8)
        def _col(d):
            dvec = jnp.full((16,), d + cbase, jnp.int32)
            for g in range(GROUPS):
                dst[0, d, pl.ds(g * 16, 16)] = \
                    plsc.load_gather(buf_g[p], [tvecs[g], dvec]) * _SCALE

    build_idx(0, 0)
    fire_gather(0)
    build_idx(1, 1)

    @pl.loop(0, SEQ // 2)
    def _pair(k):
        for so in range(2):
            s = k * 2 + so
            p = so
            pn = 1 - so

            @pl.when(s < SEQ - 1)
            def _():
                fire_gather(pn)

            wait_gather(p)

            @pl.when(s >= 1)
            def _():
                pltpu.make_async_copy(tr_a, store_dst(0, 0), sem_sa).wait()

            transpose_scale(p, 0)
            pltpu.async_copy(tr_a, store_dst(s, 0), sem_sa)

            @pl.when(s >= 1)
            def _():
                pltpu.make_async_copy(tr_b, store_dst(0, 1), sem_sb).wait()

            transpose_scale(p, 1)
            pltpu.async_copy(tr_b, store_dst(s, 1), sem_sb)

            @pl.when(s < SEQ - 2)
            def _():
                build_idx(s + 2, p)

    pltpu.make_async_copy(tr_a, store_dst(0, 0), sem_sa).wait()
    pltpu.make_async_copy(tr_b, store_dst(0, 1), sem_sb).wait()


_emb_call = pl.kernel(
    _emb_body,
    out_type=jax.ShapeDtypeStruct((SEQ, EMB_D, BATCH), jnp.float32),
    mesh=plsc.VectorSubcoreMesh(core_axis_name="c", subcore_axis_name="s"),
    compiler_params=pltpu.CompilerParams(needs_layout_passes=False),
    scratch_types=[
        pltpu.VMEM((BTOK, SEQ), jnp.int32),
        pltpu.VMEM((2, BTOK), jnp.int32),
        pltpu.VMEM((BTOK, ROW_PHYS), jnp.float32),
        pltpu.VMEM((BTOK, ROW_PHYS), jnp.float32),
        pltpu.VMEM((1, W_A, BTOK), jnp.float32),
        pltpu.VMEM((1, W_B, BTOK), jnp.float32),
        pltpu.SemaphoreType.DMA,
        pltpu.SemaphoreType.DMA,
        pltpu.SemaphoreType.DMA,
        pltpu.SemaphoreType.DMA,
    ],
)


def kernel(tokens, embedding_weight):
    out_t = _emb_call(tokens.astype(jnp.int32), embedding_weight)
    return out_t.transpose(2, 0, 1)
